# Initial kernel scaffold; baseline (speedup 1.0000x reference)
import jax
import jax.numpy as jnp
from jax.experimental import pallas as pl


def kernel(z, means, logvars, w):
    raise NotImplementedError("write your pallas kernel here")


# speedup vs baseline: 1.2723x; 1.2723x over previous
import functools
import math

import jax
import jax.numpy as jnp
from jax import lax
from jax.experimental import pallas as pl
from jax.experimental.pallas import tpu as pltpu

_B = 4096
_L = 64
_K = 256
_LANES = 128
_PACK = _LANES // _L
_ROWS = _B // _PACK
_BLOCK_ROWS = 256
_GRID = _ROWS // _BLOCK_ROWS

_C = -0.5 * math.log(2.0 * math.pi)


def _mog_body(z_ref, m_ref, lv_ref, w_ref, o_ref, a_ref, p_ref):
    z = z_ref[...]
    lv = lv_ref[...]
    w = w_ref[...]
    wmax = jnp.max(w)
    logw = w - (wmax + jnp.log(jnp.sum(jnp.exp(w - wmax))))
    a_ref[...] = (_C + logw) - 0.5 * lv
    p_ref[...] = 0.5 * jnp.exp(-lv)

    def pass1(k, mx):
        d = z - m_ref[pl.ds(k, 1), :]
        t = a_ref[pl.ds(k, 1), :] - p_ref[pl.ds(k, 1), :] * d * d
        return jnp.maximum(mx, t)

    mx = lax.fori_loop(0, _K, pass1, jnp.full(z.shape, -jnp.inf, jnp.float32),
                       unroll=8)

    def pass2(k, s):
        d = z - m_ref[pl.ds(k, 1), :]
        t = a_ref[pl.ds(k, 1), :] - p_ref[pl.ds(k, 1), :] * d * d
        return s + jnp.exp(t - mx)

    s = lax.fori_loop(0, _K, pass2, jnp.zeros(z.shape, jnp.float32),
                      unroll=8)
    o_ref[...] = mx + jnp.log(s)


@jax.jit
def kernel(z, means, logvars, w):
    z2 = z.reshape(_ROWS, _LANES)
    m2 = jnp.concatenate([means, means], axis=1)
    lv2 = jnp.concatenate([logvars, logvars], axis=1)
    wc = w.reshape(_K, 1)
    out2 = pl.pallas_call(
        _mog_body,
        grid=(_GRID,),
        in_specs=[
            pl.BlockSpec((_BLOCK_ROWS, _LANES), lambda i: (i, 0)),
            pl.BlockSpec((_K, _LANES), lambda i: (0, 0)),
            pl.BlockSpec((_K, _LANES), lambda i: (0, 0)),
            pl.BlockSpec((_K, 1), lambda i: (0, 0)),
        ],
        out_specs=pl.BlockSpec((_BLOCK_ROWS, _LANES), lambda i: (i, 0)),
        out_shape=jax.ShapeDtypeStruct((_ROWS, _LANES), jnp.float32),
        scratch_shapes=[
            pltpu.VMEM((_K, _LANES), jnp.float32),
            pltpu.VMEM((_K, _LANES), jnp.float32),
        ],
    )(z2, m2, lv2, wc)
    return out2.reshape(_B, _L)
